# baseline (speedup 1.0000x reference)
import functools

import jax
import jax.numpy as jnp
from jax import lax
from jax.experimental import pallas as pl
from jax.experimental.pallas import tpu as pltpu
from jax.experimental.pallas import tpu_sc as plsc

_N0, _N1, _N2 = 50000, 12500, 3125
_EPS = 1e-5

_SPMEM_BUDGET = 5 * 1024 * 1024 + 512 * 1024
_SB = 64
_ZR = 64


def _chunk_width(n_outp, c):
    for cc in (64, 48, 32, 16):
        if c % cc == 0 and (c // cc) >= 2 and n_outp * cc * 4 <= _SPMEM_BUDGET:
            return cc
    raise ValueError((n_outp, c))


@functools.lru_cache(maxsize=None)
def _make_sc_conv(ep, n_in, n_outp, c):
    cc = _chunk_width(n_outp, c)
    nch = c // cc
    w_total = ep // 128
    wp = w_total // 16
    assert ep % (16 * 128) == 0 and n_outp % (16 * _ZR) == 0
    nsup = -(-wp // _SB)
    rows_per_sub = n_outp // 16
    mesh = plsc.VectorSubcoreMesh(core_axis_name="c", subcore_axis_name="s")

    @functools.partial(
        pl.kernel,
        out_type=jax.ShapeDtypeStruct((nch, n_outp, cc), jnp.float32),
        mesh=mesh,
        scratch_types=[
            pltpu.VMEM((_SB, 128), jnp.int32),
            pltpu.VMEM((_SB, 128), jnp.int32),
            [pltpu.VMEM((128, cc), jnp.float32) for _ in range(4)],
            pltpu.VMEM((_ZR, cc), jnp.float32),
            pltpu.VMEM((_ZR, cc), jnp.float32),
            pltpu.VMEM_SHARED((n_outp, cc), jnp.float32),
            [pltpu.SemaphoreType.DMA for _ in range(4)],
            [pltpu.SemaphoreType.DMA for _ in range(4)],
        ],
        compiler_params=pltpu.CompilerParams(use_tc_tiling_on_sc=False),
    )
    def sc_conv(y_hbm, src_hbm, dst_hbm, zeros_hbm, out_hbm,
                srcbuf, dstbuf, rbufs, zbuf, wbuf, acc, gsems, ssems):
        cid = lax.axis_index("c")
        sid = lax.axis_index("s")
        base = sid * wp
        row0 = sid * rows_per_sub
        pltpu.sync_copy(zeros_hbm, zbuf)
        for k in range(nch):
            @pl.when(cid == (k % 2))
            def _(k=k):
                def zloop(i, carry):
                    pltpu.sync_copy(zbuf, acc.at[pl.ds(row0 + i * _ZR, _ZR)])
                    return carry
                lax.fori_loop(0, rows_per_sub // _ZR, zloop, 0)
                plsc.subcore_barrier()
                ytab = y_hbm.at[k]
                for si in range(nsup):
                    r0 = si * _SB
                    rcnt = min(_SB, wp - r0)
                    if si > 0:
                        for b in (2, 3):
                            pltpu.make_async_copy(rbufs[b],
                                                  acc.at[dstbuf.at[0]],
                                                  ssems[b]).wait()
                    pltpu.sync_copy(src_hbm.at[pl.ds(base + r0, rcnt)],
                                    srcbuf.at[pl.ds(0, rcnt)])
                    pltpu.sync_copy(dst_hbm.at[pl.ds(base + r0, rcnt)],
                                    dstbuf.at[pl.ds(0, rcnt)])
                    pltpu.async_copy(ytab.at[srcbuf.at[0]], rbufs[0], gsems[0])
                    pltpu.async_copy(ytab.at[srcbuf.at[1]], rbufs[1], gsems[1])
                    def quad(jo, carry, rcnt=rcnt):
                        for b in range(4):
                            j = jo * 4 + b
                            bn = (b + 2) % 4
                            pltpu.make_async_copy(ytab.at[srcbuf.at[j]],
                                                  rbufs[b], gsems[b]).wait()
                            pltpu.async_copy(rbufs[b], acc.at[dstbuf.at[j]],
                                             ssems[b], add=True)
                            @pl.when(j >= 2)
                            def _(j=j, bn=bn):
                                pltpu.make_async_copy(
                                    rbufs[bn], acc.at[dstbuf.at[0]],
                                    ssems[bn]).wait()
                            @pl.when(j + 2 < rcnt)
                            def _(j=j, bn=bn):
                                pltpu.async_copy(ytab.at[srcbuf.at[j + 2]],
                                                 rbufs[bn], gsems[bn])
                        return carry
                    lax.fori_loop(0, rcnt // 4, quad, 0)
                for b in (2, 3):
                    pltpu.make_async_copy(rbufs[b], acc.at[dstbuf.at[0]],
                                          ssems[b]).wait()
                plsc.subcore_barrier()
                def wbloop(i, carry):
                    r = row0 + i * _ZR
                    pltpu.sync_copy(acc.at[pl.ds(r, _ZR)], wbuf)
                    pltpu.sync_copy(wbuf, out_hbm.at[k].at[pl.ds(r, _ZR)])
                    return carry
                lax.fori_loop(0, rows_per_sub // _ZR, wbloop, 0)
                plsc.subcore_barrier()

    return sc_conv


def _sc_scatter(y, src, dst, n_out):
    n_in, c = y.shape
    e = src.shape[0]
    ep = -(-e // 16384) * 16384
    n_outp = -(-(n_out + 16) // 1024) * 1024
    pad = ep - e
    if pad:
        ar = jnp.arange(pad, dtype=jnp.int32)
        src = jnp.concatenate([src, ar % n_in])
        dst = jnp.concatenate([dst, n_out + (ar % 16)])
    src2 = src.reshape(ep // 128, 128)
    dst2 = dst.reshape(ep // 128, 128)
    cc = _chunk_width(n_outp, c)
    nch = c // cc
    ych = y.reshape(n_in, nch, cc).transpose(1, 0, 2)
    zeros = jnp.zeros((_ZR, cc), jnp.float32)
    out = _make_sc_conv(ep, n_in, n_outp, c)(ych, src2, dst2, zeros)
    return out.transpose(1, 0, 2).reshape(n_outp, c)[:n_out]


def _fused_mm(x, W, scale=None, shift=None, resid=None, act=False, out_bias=None):
    N, Ci = x.shape
    Co = W.shape[1]
    BR = 512
    grid = (pl.cdiv(N, BR),)
    has_ss = scale is not None
    has_r = resid is not None
    has_b = out_bias is not None

    def body(*refs):
        x_ref, w_ref = refs[0], refs[1]
        k = 2
        t = x_ref[...]
        if has_ss:
            t = t * refs[k][...] + refs[k + 1][...]
            k += 2
        if has_r:
            t = t + refs[k][...]
            k += 1
        if act:
            t = jnp.maximum(t, 0.0)
        y = jnp.dot(t, w_ref[...], preferred_element_type=jnp.float32)
        if has_b:
            y = y + refs[k][...]
            k += 1
        refs[k][...] = y

    in_specs = [
        pl.BlockSpec((BR, Ci), lambda i: (i, 0)),
        pl.BlockSpec((Ci, Co), lambda i: (0, 0)),
    ]
    args = [x, W]
    if has_ss:
        in_specs += [pl.BlockSpec((1, Ci), lambda i: (0, 0))] * 2
        args += [scale.reshape(1, Ci), shift.reshape(1, Ci)]
    if has_r:
        in_specs.append(pl.BlockSpec((BR, Ci), lambda i: (i, 0)))
        args.append(resid)
    if has_b:
        in_specs.append(pl.BlockSpec((1, Co), lambda i: (0, 0)))
        args.append(out_bias.reshape(1, Co))
    return pl.pallas_call(
        body,
        grid=grid,
        in_specs=in_specs,
        out_specs=pl.BlockSpec((BR, Co), lambda i: (i, 0)),
        out_shape=jax.ShapeDtypeStruct((N, Co), jnp.float32),
        compiler_params=pltpu.CompilerParams(
            dimension_semantics=("arbitrary",)),
    )(*args)


def _bn(h, g, b):
    m = jnp.mean(h, axis=0)
    v = jnp.var(h, axis=0)
    return (h - m) / jnp.sqrt(v + _EPS) * g + b


def _scatter(y, src, dst, n):
    return _sc_scatter(y, src, dst, n)


def _conv(x, W, src, dst, n):
    return _scatter(_fused_mm(x, W), src, dst, n)


def _cbr(x, p, src, dst, n):
    return jnp.maximum(_bn(_conv(x, p["W"], src, dst, n), p["g"], p["b"]), 0.0)


def _res(x, p, src, dst, n):
    h = jnp.maximum(_bn(_conv(x, p["W1"], src, dst, n), p["g1"], p["b1"]), 0.0)
    h = _bn(_conv(h, p["W2"], src, dst, n), p["g2"], p["b2"])
    sc = _bn(_fused_mm(x, p["Wd"]), p["gd"], p["bd"]) if "Wd" in p else x
    return jnp.maximum(h + sc, 0.0)


def kernel(x, edge0, edge1, edge2, down01_src, down01_dst, down12_src, down12_dst, params):
    p = params
    x0 = _cbr(x, p["stem1"], edge0[0], edge0[1], _N0)
    x0 = _cbr(x0, p["stem2"], edge0[0], edge0[1], _N0)
    x1 = _cbr(x0, p["enc0_down"], down01_src, down01_dst, _N1)
    x1 = _res(x1, p["enc0_res1"], edge1[0], edge1[1], _N1)
    x1 = _res(x1, p["enc0_res2"], edge1[0], edge1[1], _N1)
    x2 = _cbr(x1, p["enc1_down"], down12_src, down12_dst, _N2)
    x2 = _res(x2, p["enc1_res1"], edge2[0], edge2[1], _N2)
    x2 = _res(x2, p["enc1_res2"], edge2[0], edge2[1], _N2)
    y = _cbr(x2, p["dec0_up"], down12_dst, down12_src, _N1)
    y = jnp.concatenate([y, x1], axis=1)
    y = _res(y, p["dec0_res1"], edge1[0], edge1[1], _N1)
    y = _res(y, p["dec0_res2"], edge1[0], edge1[1], _N1)
    y = _cbr(y, p["dec1_up"], down01_dst, down01_src, _N0)
    y = jnp.concatenate([y, x0], axis=1)
    y = _res(y, p["dec1_res1"], edge0[0], edge0[1], _N0)
    y = _res(y, p["dec1_res2"], edge0[0], edge0[1], _N0)
    return _fused_mm(y, p["cls_W"], out_bias=p["cls_b"])


# speedup vs baseline: 1.1602x; 1.1602x over previous
import functools

import jax
import jax.numpy as jnp
from jax import lax
from jax.experimental import pallas as pl
from jax.experimental.pallas import tpu as pltpu
from jax.experimental.pallas import tpu_sc as plsc

_N0, _N1, _N2 = 50000, 12500, 3125
_EPS = 1e-5

_SPMEM_BUDGET = 7 * 1024 * 1024
_SB = 64
_ZR = 64


def _chunk_width(n_outp, c):
    for cc in (64, 48, 32, 16):
        if c % cc == 0 and (c // cc) >= 2 and n_outp * cc * 4 <= _SPMEM_BUDGET:
            return cc
    raise ValueError((n_outp, c))


@functools.lru_cache(maxsize=None)
def _make_sc_conv(ep, n_in, n_outp, c):
    cc = _chunk_width(n_outp, c)
    nch = c // cc
    w_total = ep // 128
    wp = w_total // 16
    assert ep % (16 * 128) == 0 and n_outp % (16 * _ZR) == 0
    nsup = -(-wp // _SB)
    rows_per_sub = n_outp // 16
    mesh = plsc.VectorSubcoreMesh(core_axis_name="c", subcore_axis_name="s")

    @functools.partial(
        pl.kernel,
        out_type=jax.ShapeDtypeStruct((nch, n_outp, cc), jnp.float32),
        mesh=mesh,
        scratch_types=[
            pltpu.VMEM((_SB, 128), jnp.int32),
            pltpu.VMEM((_SB, 128), jnp.int32),
            [pltpu.VMEM((128, cc), jnp.float32) for _ in range(2)],
            pltpu.VMEM((_ZR, cc), jnp.float32),
            pltpu.VMEM((_ZR, cc), jnp.float32),
            pltpu.VMEM_SHARED((n_outp, cc), jnp.float32),
            [pltpu.SemaphoreType.DMA for _ in range(2)],
        ],
        compiler_params=pltpu.CompilerParams(use_tc_tiling_on_sc=False),
    )
    def sc_conv(y_hbm, src_hbm, dst_hbm, zeros_hbm, out_hbm,
                srcbuf, dstbuf, rbufs, zbuf, wbuf, acc, gsems):
        cid = lax.axis_index("c")
        sid = lax.axis_index("s")
        base = sid * wp
        row0 = sid * rows_per_sub
        pltpu.sync_copy(zeros_hbm, zbuf)
        for k in range(nch):
            @pl.when(cid == (k % 2))
            def _(k=k):
                def zloop(i, carry):
                    pltpu.sync_copy(zbuf, acc.at[pl.ds(row0 + i * _ZR, _ZR)])
                    return carry
                lax.fori_loop(0, rows_per_sub // _ZR, zloop, 0)
                plsc.subcore_barrier()
                ytab = y_hbm.at[k]
                for si in range(nsup):
                    r0 = si * _SB
                    rcnt = min(_SB, wp - r0)
                    pltpu.sync_copy(src_hbm.at[pl.ds(base + r0, rcnt)],
                                    srcbuf.at[pl.ds(0, rcnt)])
                    pltpu.sync_copy(dst_hbm.at[pl.ds(base + r0, rcnt)],
                                    dstbuf.at[pl.ds(0, rcnt)])
                    pltpu.async_copy(ytab.at[srcbuf.at[0]], rbufs[0], gsems[0])
                    pltpu.async_copy(ytab.at[srcbuf.at[1]], rbufs[1], gsems[1])
                    def pair(jo, carry):
                        for b in range(2):
                            j = jo * 2 + b
                            pltpu.make_async_copy(ytab.at[srcbuf.at[j]],
                                                  rbufs[b], gsems[b]).wait()
                            pltpu.sync_copy(rbufs[b], acc.at[dstbuf.at[j]],
                                            add=True)
                            pltpu.async_copy(ytab.at[srcbuf.at[j + 2]],
                                             rbufs[b], gsems[b])
                        return carry
                    lax.fori_loop(0, rcnt // 2 - 1, pair, 0)
                    for b in range(2):
                        j = rcnt - 2 + b
                        pltpu.make_async_copy(ytab.at[srcbuf.at[j]],
                                              rbufs[b], gsems[b]).wait()
                        pltpu.sync_copy(rbufs[b], acc.at[dstbuf.at[j]],
                                        add=True)
                plsc.subcore_barrier()
                def wbloop(i, carry):
                    r = row0 + i * _ZR
                    pltpu.sync_copy(acc.at[pl.ds(r, _ZR)], wbuf)
                    pltpu.sync_copy(wbuf, out_hbm.at[k].at[pl.ds(r, _ZR)])
                    return carry
                lax.fori_loop(0, rows_per_sub // _ZR, wbloop, 0)
                plsc.subcore_barrier()

    return sc_conv


def _sc_scatter(y, src, dst, n_out):
    n_in, c = y.shape
    e = src.shape[0]
    ep = -(-e // 16384) * 16384
    n_outp = -(-(n_out + 16) // 1024) * 1024
    pad = ep - e
    if pad:
        ar = jnp.arange(pad, dtype=jnp.int32)
        src = jnp.concatenate([src, ar % n_in])
        dst = jnp.concatenate([dst, n_out + (ar % 16)])
    src2 = src.reshape(ep // 128, 128)
    dst2 = dst.reshape(ep // 128, 128)
    cc = _chunk_width(n_outp, c)
    nch = c // cc
    ych = y.reshape(n_in, nch, cc).transpose(1, 0, 2)
    zeros = jnp.zeros((_ZR, cc), jnp.float32)
    out = _make_sc_conv(ep, n_in, n_outp, c)(ych, src2, dst2, zeros)
    return out.transpose(1, 0, 2).reshape(n_outp, c)[:n_out]


def _fused_mm(x, W, scale=None, shift=None, resid=None, act=False, out_bias=None):
    N, Ci = x.shape
    Co = W.shape[1]
    BR = 512
    grid = (pl.cdiv(N, BR),)
    has_ss = scale is not None
    has_r = resid is not None
    has_b = out_bias is not None

    def body(*refs):
        x_ref, w_ref = refs[0], refs[1]
        k = 2
        t = x_ref[...]
        if has_ss:
            t = t * refs[k][...] + refs[k + 1][...]
            k += 2
        if has_r:
            t = t + refs[k][...]
            k += 1
        if act:
            t = jnp.maximum(t, 0.0)
        y = jnp.dot(t, w_ref[...], preferred_element_type=jnp.float32)
        if has_b:
            y = y + refs[k][...]
            k += 1
        refs[k][...] = y

    in_specs = [
        pl.BlockSpec((BR, Ci), lambda i: (i, 0)),
        pl.BlockSpec((Ci, Co), lambda i: (0, 0)),
    ]
    args = [x, W]
    if has_ss:
        in_specs += [pl.BlockSpec((1, Ci), lambda i: (0, 0))] * 2
        args += [scale.reshape(1, Ci), shift.reshape(1, Ci)]
    if has_r:
        in_specs.append(pl.BlockSpec((BR, Ci), lambda i: (i, 0)))
        args.append(resid)
    if has_b:
        in_specs.append(pl.BlockSpec((1, Co), lambda i: (0, 0)))
        args.append(out_bias.reshape(1, Co))
    return pl.pallas_call(
        body,
        grid=grid,
        in_specs=in_specs,
        out_specs=pl.BlockSpec((BR, Co), lambda i: (i, 0)),
        out_shape=jax.ShapeDtypeStruct((N, Co), jnp.float32),
        compiler_params=pltpu.CompilerParams(
            dimension_semantics=("arbitrary",)),
    )(*args)


def _bn(h, g, b):
    m = jnp.mean(h, axis=0)
    v = jnp.var(h, axis=0)
    return (h - m) / jnp.sqrt(v + _EPS) * g + b


def _scatter(y, src, dst, n):
    return _sc_scatter(y, src, dst, n)


def _conv(x, W, src, dst, n):
    return _scatter(_fused_mm(x, W), src, dst, n)


def _cbr(x, p, src, dst, n):
    return jnp.maximum(_bn(_conv(x, p["W"], src, dst, n), p["g"], p["b"]), 0.0)


def _res(x, p, src, dst, n):
    h = jnp.maximum(_bn(_conv(x, p["W1"], src, dst, n), p["g1"], p["b1"]), 0.0)
    h = _bn(_conv(h, p["W2"], src, dst, n), p["g2"], p["b2"])
    sc = _bn(_fused_mm(x, p["Wd"]), p["gd"], p["bd"]) if "Wd" in p else x
    return jnp.maximum(h + sc, 0.0)


def kernel(x, edge0, edge1, edge2, down01_src, down01_dst, down12_src, down12_dst, params):
    p = params
    x0 = _cbr(x, p["stem1"], edge0[0], edge0[1], _N0)
    x0 = _cbr(x0, p["stem2"], edge0[0], edge0[1], _N0)
    x1 = _cbr(x0, p["enc0_down"], down01_src, down01_dst, _N1)
    x1 = _res(x1, p["enc0_res1"], edge1[0], edge1[1], _N1)
    x1 = _res(x1, p["enc0_res2"], edge1[0], edge1[1], _N1)
    x2 = _cbr(x1, p["enc1_down"], down12_src, down12_dst, _N2)
    x2 = _res(x2, p["enc1_res1"], edge2[0], edge2[1], _N2)
    x2 = _res(x2, p["enc1_res2"], edge2[0], edge2[1], _N2)
    y = _cbr(x2, p["dec0_up"], down12_dst, down12_src, _N1)
    y = jnp.concatenate([y, x1], axis=1)
    y = _res(y, p["dec0_res1"], edge1[0], edge1[1], _N1)
    y = _res(y, p["dec0_res2"], edge1[0], edge1[1], _N1)
    y = _cbr(y, p["dec1_up"], down01_dst, down01_src, _N0)
    y = jnp.concatenate([y, x0], axis=1)
    y = _res(y, p["dec1_res1"], edge0[0], edge0[1], _N0)
    y = _res(y, p["dec1_res2"], edge0[0], edge0[1], _N0)
    return _fused_mm(y, p["cls_W"], out_bias=p["cls_b"])
